# Initial kernel scaffold; baseline (speedup 1.0000x reference)
import jax
import jax.numpy as jnp
from jax import lax
from jax.experimental import pallas as pl
from jax.experimental.pallas import tpu as pltpu
from jax.experimental.pallas import tpu_sc as plsc

NC = 2
NS = 16
NW = NC * NS
LANES = 16
CH = 128
NBUF = 3

_NP = 10240
_F = 128
_RPT = _NP // NS
_PPT = _RPT // CH


def _mesh():
    return plsc.VectorSubcoreMesh(
        core_axis_name="c", subcore_axis_name="s",
        num_cores=NC, num_subcores=NS)


def _sc_degree(dst_r):
    nchunk = dst_r.shape[1]

    def body(dst_hbm, out_hbm, dst_v, ones_v, zbuf, acc):
        cid = lax.axis_index("c")
        sid = lax.axis_index("s")
        wid = sid * NC + cid
        pltpu.sync_copy(dst_hbm.at[wid], dst_v)

        def fill_ones(i, c):
            ones_v[i] = jnp.full((LANES,), 1.0, jnp.float32)
            return c
        lax.fori_loop(0, CH, fill_ones, 0)

        def fill_zero(i, c):
            zbuf[i] = jnp.zeros((LANES,), jnp.float32)
            return c
        lax.fori_loop(0, _RPT, fill_zero, 0)
        pltpu.sync_copy(zbuf, acc.at[pl.ds(sid * _RPT, _RPT)])
        plsc.subcore_barrier()

        def step(j, c):
            pltpu.sync_copy(ones_v, acc.at[dst_v.at[j]], add=True)
            return c
        lax.fori_loop(0, nchunk, step, 0)
        plsc.subcore_barrier()

        pltpu.sync_copy(acc.at[pl.ds(sid * _RPT, _RPT)], zbuf)
        pltpu.sync_copy(zbuf, out_hbm.at[pl.ds(cid * _NP + sid * _RPT, _RPT)])

    return pl.kernel(
        body,
        out_type=jax.ShapeDtypeStruct((NC * _NP, LANES), jnp.float32),
        mesh=_mesh(),
        scratch_types=[
            pltpu.VMEM((nchunk, CH), jnp.int32),
            pltpu.VMEM((CH, LANES), jnp.float32),
            pltpu.VMEM((_RPT, LANES), jnp.float32),
            pltpu.VMEM_SHARED((_NP, LANES), jnp.float32),
        ],
    )(dst_r)


def _sc_scatter(u_pad, src_r, dst_r, zz):
    nchunk = src_r.shape[1]
    outer = nchunk // NBUF

    def body(u_hbm, src_hbm, dst_hbm, zz_hbm, out_hbm,
             src_v, dst_v, rows, acc, s0, s1, s2):
        cid = lax.axis_index("c")
        sid = lax.axis_index("s")
        wid = sid * NC + cid
        pltpu.sync_copy(src_hbm.at[wid], src_v)
        pltpu.sync_copy(dst_hbm.at[wid], dst_v)

        pltpu.sync_copy(zz_hbm.at[pl.ds(0, CH)], rows.at[0])
        for p in range(_PPT):
            pltpu.sync_copy(rows.at[0],
                            acc.at[pl.ds(sid * _RPT + p * CH, CH)])
        plsc.subcore_barrier()

        sems = (s0, s1, s2)

        def step(i, c):
            j0 = i * NBUF
            descs = []
            for b in range(NBUF):
                descs.append(pltpu.async_copy(
                    u_hbm.at[src_v.at[j0 + b]], rows.at[b], sems[b]))
            for b in range(NBUF):
                descs[b].wait()
                pltpu.sync_copy(rows.at[b], acc.at[dst_v.at[j0 + b]],
                                add=True)
            return c
        lax.fori_loop(0, outer, step, 0)
        plsc.subcore_barrier()

        base = cid * _NP + sid * _RPT
        for p in range(_PPT):
            pltpu.sync_copy(acc.at[pl.ds(sid * _RPT + p * CH, CH)],
                            rows.at[0])
            pltpu.sync_copy(rows.at[0], out_hbm.at[pl.ds(base + p * CH, CH)])

    return pl.kernel(
        body,
        out_type=jax.ShapeDtypeStruct((NC * _NP, _F), jnp.float32),
        mesh=_mesh(),
        scratch_types=[
            pltpu.VMEM((nchunk, CH), jnp.int32),
            pltpu.VMEM((nchunk, CH), jnp.int32),
            pltpu.VMEM((NBUF, CH, _F), jnp.float32),
            pltpu.VMEM_SHARED((_NP, _F), jnp.float32),
            pltpu.SemaphoreType.DMA,
            pltpu.SemaphoreType.DMA,
            pltpu.SemaphoreType.DMA,
        ],
    )(u_pad, src_r, dst_r, zz)


_R = 1280


def _tc_a(d0, d1, x_pad):
    def body(d0_ref, d1_ref, x_ref, dis_ref, u1_ref):
        deg = d0_ref[:, 0:1] + d1_ref[:, 0:1]
        dis = lax.rsqrt(deg)
        dis_ref[...] = jnp.broadcast_to(dis, (_R, _F))
        u1_ref[...] = x_ref[...] * dis

    grid = _NP // _R
    return pl.pallas_call(
        body,
        grid=(grid,),
        in_specs=[
            pl.BlockSpec((_R, LANES), lambda i: (i, 0)),
            pl.BlockSpec((_R, LANES), lambda i: (i, 0)),
            pl.BlockSpec((_R, _F), lambda i: (i, 0)),
        ],
        out_specs=[
            pl.BlockSpec((_R, _F), lambda i: (i, 0)),
            pl.BlockSpec((_R, _F), lambda i: (i, 0)),
        ],
        out_shape=[
            jax.ShapeDtypeStruct((_NP, _F), jnp.float32),
            jax.ShapeDtypeStruct((_NP, _F), jnp.float32),
        ],
    )(d0, d1, x_pad)


def _tc_b(s0, s1, disb, W1, b1r):
    def body(s0_ref, s1_ref, dis_ref, w_ref, b_ref, u2_ref):
        agg = dis_ref[...] * (s0_ref[...] + s1_ref[...])
        h = jnp.dot(agg, w_ref[...],
                    preferred_element_type=jnp.float32,
                    precision=lax.Precision.HIGHEST) + b_ref[...]
        u2_ref[...] = dis_ref[...] * jnp.maximum(h, 0.0)

    grid = _NP // _R
    hid = W1.shape[1]
    return pl.pallas_call(
        body,
        grid=(grid,),
        in_specs=[
            pl.BlockSpec((_R, _F), lambda i: (i, 0)),
            pl.BlockSpec((_R, _F), lambda i: (i, 0)),
            pl.BlockSpec((_R, _F), lambda i: (i, 0)),
            pl.BlockSpec((_F, hid), lambda i: (0, 0)),
            pl.BlockSpec((1, hid), lambda i: (0, 0)),
        ],
        out_specs=pl.BlockSpec((_R, hid), lambda i: (i, 0)),
        out_shape=jax.ShapeDtypeStruct((_NP, hid), jnp.float32),
    )(s0, s1, disb, W1, b1r)


def _tc_c(s0, s1, disb, Wmu, bmur, Wls, blsr):
    oc = Wmu.shape[1]

    def body(s0_ref, s1_ref, dis_ref, wm_ref, bm_ref, wl_ref, bl_ref,
             mu_ref, ls_ref):
        agg = dis_ref[...] * (s0_ref[...] + s1_ref[...])
        mu_ref[...] = jnp.dot(agg, wm_ref[...],
                              preferred_element_type=jnp.float32,
                              precision=lax.Precision.HIGHEST) + bm_ref[...]
        ls_ref[...] = jnp.dot(agg, wl_ref[...],
                              preferred_element_type=jnp.float32,
                              precision=lax.Precision.HIGHEST) + bl_ref[...]

    grid = _NP // _R
    return pl.pallas_call(
        body,
        grid=(grid,),
        in_specs=[
            pl.BlockSpec((_R, _F), lambda i: (i, 0)),
            pl.BlockSpec((_R, _F), lambda i: (i, 0)),
            pl.BlockSpec((_R, _F), lambda i: (i, 0)),
            pl.BlockSpec((_F, oc), lambda i: (0, 0)),
            pl.BlockSpec((1, oc), lambda i: (0, 0)),
            pl.BlockSpec((_F, oc), lambda i: (0, 0)),
            pl.BlockSpec((1, oc), lambda i: (0, 0)),
        ],
        out_specs=[
            pl.BlockSpec((_R, oc), lambda i: (i, 0)),
            pl.BlockSpec((_R, oc), lambda i: (i, 0)),
        ],
        out_shape=[
            jax.ShapeDtypeStruct((_NP, oc), jnp.float32),
            jax.ShapeDtypeStruct((_NP, oc), jnp.float32),
        ],
    )(s0, s1, disb, Wmu, bmur, Wls, blsr)


def kernel(x, edge_index, W1, b1, Wmu, bmu, Wls, bls):
    n = x.shape[0]
    e = edge_index.shape[1]
    x_pad = jnp.pad(x, ((0, _NP - n), (0, 0)))

    loops = jnp.arange(_NP, dtype=edge_index.dtype)
    src = jnp.concatenate([edge_index[0], loops])
    dst = jnp.concatenate([edge_index[1], loops])
    etot = e + _NP
    nchunk = -(-etot // (NW * CH))
    nchunk = ((nchunk + NBUF - 1) // NBUF) * NBUF
    ep = nchunk * NW * CH
    src_r = jnp.pad(src, (0, ep - etot), constant_values=n).reshape(
        NW, nchunk, CH)
    dst_r = jnp.pad(dst, (0, ep - etot), constant_values=n).reshape(
        NW, nchunk, CH)
    zz = jnp.zeros((_NP, _F), jnp.float32)

    degp = _sc_degree(dst_r)
    disb, u1 = _tc_a(degp[:_NP], degp[_NP:], x_pad)
    s1 = _sc_scatter(u1, src_r, dst_r, zz)
    u2 = _tc_b(s1[:_NP], s1[_NP:], disb, W1, b1.reshape(1, -1))
    s2 = _sc_scatter(u2, src_r, dst_r, zz)
    mu, ls = _tc_c(s2[:_NP], s2[_NP:], disb, Wmu, bmu.reshape(1, -1),
                   Wls, bls.reshape(1, -1))
    return mu[:n], ls[:n]


# speedup vs baseline: 19.4720x; 19.4720x over previous
import jax
import jax.numpy as jnp
from jax import lax
from jax.experimental import pallas as pl
from jax.experimental.pallas import tpu as pltpu
from jax.experimental.pallas import tpu_sc as plsc

NC = 2
NS = 16
NW = NC * NS
LANES = 16
CH = 128
NBUF = 3

_NP = 10240
_F = 128
_RPT = _NP // NS
_PPT = _RPT // CH


def _mesh():
    return plsc.VectorSubcoreMesh(
        core_axis_name="c", subcore_axis_name="s",
        num_cores=NC, num_subcores=NS)


def _sc_degree(dst_r, zz):
    nchunk = dst_r.shape[1]

    def body(dst_hbm, zz_hbm, out_hbm, dst_v, ones_v, acc):
        cid = lax.axis_index("c")
        sid = lax.axis_index("s")
        wid = sid * NC + cid
        pltpu.sync_copy(dst_hbm.at[wid], dst_v)

        pltpu.sync_copy(zz_hbm.at[pl.ds(0, CH)], ones_v)
        for p in range(_PPT):
            pltpu.sync_copy(ones_v, acc.at[pl.ds(sid * _RPT + p * CH, CH)])

        def fill_ones(r, c):
            for q in range(CH // LANES):
                ones_v[r, pl.ds(q * LANES, LANES)] = jnp.full(
                    (LANES,), 1.0, jnp.float32)
            return c
        lax.fori_loop(0, CH, fill_ones, 0)
        plsc.subcore_barrier()

        def step(j, c):
            pltpu.sync_copy(ones_v, acc.at[dst_v.at[j]], add=True)
            return c
        lax.fori_loop(0, nchunk, step, 0)
        plsc.subcore_barrier()

        base = cid * _NP + sid * _RPT
        for p in range(_PPT):
            pltpu.sync_copy(acc.at[pl.ds(sid * _RPT + p * CH, CH)], ones_v)
            pltpu.sync_copy(ones_v, out_hbm.at[pl.ds(base + p * CH, CH)])

    return pl.kernel(
        body,
        out_type=jax.ShapeDtypeStruct((NC * _NP, CH), jnp.float32),
        mesh=_mesh(),
        scratch_types=[
            pltpu.VMEM((nchunk, CH), jnp.int32),
            pltpu.VMEM((CH, CH), jnp.float32),
            pltpu.VMEM_SHARED((_NP, CH), jnp.float32),
        ],
    )(dst_r, zz)


CHS = 96
F0 = 152
F1 = 64
_WB = (6, 64)


def _sc_scatter(u_pad, pk0, pk1, zz):
    def body(u_hbm, pk0_hbm, pk1_hbm, zz_hbm, out_hbm,
             pk_v, sbuf, dbuf, rows, acc, g0, g1):
        cid = lax.axis_index("c")
        sid = lax.axis_index("s")

        @pl.when(cid == 0)
        def _stage0():
            pltpu.sync_copy(pk0_hbm.at[sid], pk_v)

        @pl.when(cid == 1)
        def _stage1():
            pltpu.sync_copy(pk1_hbm.at[sid], pk_v.at[pl.ds(0, F1)])

        pltpu.sync_copy(zz_hbm.at[pl.ds(0, CHS)], rows.at[0])
        for p in range(_WB[0]):
            pltpu.sync_copy(rows.at[0],
                            acc.at[pl.ds(sid * _RPT + p * CHS, CHS)])
        pltpu.sync_copy(rows.at[0].at[pl.ds(0, _WB[1])],
                        acc.at[pl.ds(sid * _RPT + _WB[0] * CHS, _WB[1])])
        plsc.subcore_barrier()

        gsems = (g0, g1)

        def unpack(c, b):
            for q in range(CHS // LANES):
                v = pk_v[c, pl.ds(q * LANES, LANES)]
                sbuf[b, pl.ds(q * LANES, LANES)] = v & jnp.int32(0xFFFF)
                dbuf[b, pl.ds(q * LANES, LANES)] = lax.shift_right_logical(
                    v, jnp.int32(16))

        def step(i, c):
            c0 = i * 2
            descs = []
            for b in range(2):
                unpack(c0 + b, b)
                descs.append(pltpu.async_copy(
                    u_hbm.at[sbuf.at[b]], rows.at[b], gsems[b]))
            for b in range(2):
                descs[b].wait()
                pltpu.sync_copy(rows.at[b], acc.at[dbuf.at[b]], add=True)
            return c
        npr = jnp.where(cid == 0, F0 // 2, F1 // 2)
        lax.fori_loop(0, npr, step, 0)
        plsc.subcore_barrier()

        base = cid * _NP + sid * _RPT
        for p in range(_WB[0]):
            pltpu.sync_copy(acc.at[pl.ds(sid * _RPT + p * CHS, CHS)],
                            rows.at[0])
            pltpu.sync_copy(rows.at[0], out_hbm.at[pl.ds(base + p * CHS,
                                                         CHS)])
        pltpu.sync_copy(acc.at[pl.ds(sid * _RPT + _WB[0] * CHS, _WB[1])],
                        rows.at[0].at[pl.ds(0, _WB[1])])
        pltpu.sync_copy(rows.at[0].at[pl.ds(0, _WB[1])],
                        out_hbm.at[pl.ds(base + _WB[0] * CHS, _WB[1])])

    return pl.kernel(
        body,
        out_type=jax.ShapeDtypeStruct((NC * _NP, _F), jnp.float32),
        mesh=_mesh(),
        scratch_types=[
            pltpu.VMEM((F0, CHS), jnp.int32),
            pltpu.VMEM((2, CHS), jnp.int32),
            pltpu.VMEM((2, CHS), jnp.int32),
            pltpu.VMEM((2, CHS, _F), jnp.float32),
            pltpu.VMEM_SHARED((_NP, _F), jnp.float32),
            pltpu.SemaphoreType.DMA,
            pltpu.SemaphoreType.DMA,
        ],
    )(u_pad, pk0, pk1, zz)


_R = 1280


def _tc_a(d0, d1, x_pad):
    def body(d0_ref, d1_ref, x_ref, dis_ref, u1_ref):
        dis = lax.rsqrt(d0_ref[...] + d1_ref[...])
        dis_ref[...] = dis
        u1_ref[...] = x_ref[...] * dis

    grid = _NP // _R
    return pl.pallas_call(
        body,
        grid=(grid,),
        in_specs=[
            pl.BlockSpec((_R, _F), lambda i: (i, 0)),
            pl.BlockSpec((_R, _F), lambda i: (i, 0)),
            pl.BlockSpec((_R, _F), lambda i: (i, 0)),
        ],
        out_specs=[
            pl.BlockSpec((_R, _F), lambda i: (i, 0)),
            pl.BlockSpec((_R, _F), lambda i: (i, 0)),
        ],
        out_shape=[
            jax.ShapeDtypeStruct((_NP, _F), jnp.float32),
            jax.ShapeDtypeStruct((_NP, _F), jnp.float32),
        ],
    )(d0, d1, x_pad)


def _tc_b(s0, s1, disb, W1, b1r):
    def body(s0_ref, s1_ref, dis_ref, w_ref, b_ref, u2_ref):
        agg = dis_ref[...] * (s0_ref[...] + s1_ref[...])
        h = jnp.dot(agg, w_ref[...],
                    preferred_element_type=jnp.float32,
                    precision=lax.Precision.HIGHEST) + b_ref[...]
        u2_ref[...] = dis_ref[...] * jnp.maximum(h, 0.0)

    grid = _NP // _R
    hid = W1.shape[1]
    return pl.pallas_call(
        body,
        grid=(grid,),
        in_specs=[
            pl.BlockSpec((_R, _F), lambda i: (i, 0)),
            pl.BlockSpec((_R, _F), lambda i: (i, 0)),
            pl.BlockSpec((_R, _F), lambda i: (i, 0)),
            pl.BlockSpec((_F, hid), lambda i: (0, 0)),
            pl.BlockSpec((1, hid), lambda i: (0, 0)),
        ],
        out_specs=pl.BlockSpec((_R, hid), lambda i: (i, 0)),
        out_shape=jax.ShapeDtypeStruct((_NP, hid), jnp.float32),
    )(s0, s1, disb, W1, b1r)


def _tc_c(s0, s1, disb, Wmu, bmur, Wls, blsr):
    oc = Wmu.shape[1]

    def body(s0_ref, s1_ref, dis_ref, wm_ref, bm_ref, wl_ref, bl_ref,
             mu_ref, ls_ref):
        agg = dis_ref[...] * (s0_ref[...] + s1_ref[...])
        mu_ref[...] = jnp.dot(agg, wm_ref[...],
                              preferred_element_type=jnp.float32,
                              precision=lax.Precision.HIGHEST) + bm_ref[...]
        ls_ref[...] = jnp.dot(agg, wl_ref[...],
                              preferred_element_type=jnp.float32,
                              precision=lax.Precision.HIGHEST) + bl_ref[...]

    grid = _NP // _R
    return pl.pallas_call(
        body,
        grid=(grid,),
        in_specs=[
            pl.BlockSpec((_R, _F), lambda i: (i, 0)),
            pl.BlockSpec((_R, _F), lambda i: (i, 0)),
            pl.BlockSpec((_R, _F), lambda i: (i, 0)),
            pl.BlockSpec((_F, oc), lambda i: (0, 0)),
            pl.BlockSpec((1, oc), lambda i: (0, 0)),
            pl.BlockSpec((_F, oc), lambda i: (0, 0)),
            pl.BlockSpec((1, oc), lambda i: (0, 0)),
        ],
        out_specs=[
            pl.BlockSpec((_R, oc), lambda i: (i, 0)),
            pl.BlockSpec((_R, oc), lambda i: (i, 0)),
        ],
        out_shape=[
            jax.ShapeDtypeStruct((_NP, oc), jnp.float32),
            jax.ShapeDtypeStruct((_NP, oc), jnp.float32),
        ],
    )(s0, s1, disb, Wmu, bmur, Wls, blsr)


def kernel(x, edge_index, W1, b1, Wmu, bmu, Wls, bls):
    n = x.shape[0]
    e = edge_index.shape[1]
    x_pad = jnp.pad(x, ((0, _NP - n), (0, 0)))

    loops = jnp.arange(_NP, dtype=edge_index.dtype)
    src = jnp.concatenate([edge_index[0], loops])
    dst = jnp.concatenate([edge_index[1], loops])
    etot = e + _NP
    nchunk = -(-etot // (NW * CH))
    ep = nchunk * NW * CH
    dst_r = jnp.pad(dst, (0, ep - etot), constant_values=n).reshape(
        NW, nchunk, CH)
    eps = (16 * (F0 + F1)) * CHS
    pk = (jnp.pad(src, (0, eps - etot), constant_values=n)
          | (jnp.pad(dst, (0, eps - etot), constant_values=n) << 16))
    pk0 = pk[:16 * F0 * CHS].reshape(NS, F0, CHS)
    pk1 = pk[16 * F0 * CHS:].reshape(NS, F1, CHS)
    zz = jnp.zeros((_NP, _F), jnp.float32)

    degp = _sc_degree(dst_r, zz)
    disb, u1 = _tc_a(degp[:_NP], degp[_NP:], x_pad)
    s1 = _sc_scatter(u1, pk0, pk1, zz)
    u2 = _tc_b(s1[:_NP], s1[_NP:], disb, W1, b1.reshape(1, -1))
    s2 = _sc_scatter(u2, pk0, pk1, zz)
    mu, ls = _tc_c(s2[:_NP], s2[_NP:], disb, Wmu, bmu.reshape(1, -1),
                   Wls, bls.reshape(1, -1))
    return mu[:n], ls[:n]
